# baseline (speedup 1.0000x reference)
import functools

import jax
import jax.numpy as jnp
from jax import lax
from jax.experimental import pallas as pl
from jax.experimental.pallas import tpu as pltpu
from jax.experimental.pallas import tpu_sc as plsc

_VOCAB = 1000000
_EMBED = 64
_LPAD = 16
_PACK = 8
_ROW_BLK = 1000


def _transform_body(t_ref, w_ref, b_ref, o_ref):
    x = jnp.tanh(t_ref[...])
    o_ref[...] = (
        jnp.dot(x, w_ref[...], preferred_element_type=jnp.float32) + b_ref[...]
    )


def _transform_table(tbl8, w8, b8):
    rows = tbl8.shape[0]
    k = tbl8.shape[1]
    n = w8.shape[1]
    return pl.pallas_call(
        _transform_body,
        grid=(rows // _ROW_BLK,),
        in_specs=[
            pl.BlockSpec((_ROW_BLK, k), lambda i: (i, 0)),
            pl.BlockSpec((k, n), lambda i: (0, 0)),
            pl.BlockSpec((1, n), lambda i: (0, 0)),
        ],
        out_specs=pl.BlockSpec((_ROW_BLK, n), lambda i: (i, 0)),
        out_shape=jax.ShapeDtypeStruct((rows, n), jnp.float32),
    )(tbl8, w8, b8)


_CHUNK = 128
_NBUF = 4


def _sc_gather(table, idx):
    info = plsc.get_sparse_core_info()
    nc, ns = info.num_cores, info.num_subcores
    nw = nc * ns
    b_total = idx.shape[0]
    b_per_w = b_total // nw
    n_steps = b_per_w // _CHUNK
    n_groups = n_steps // _NBUF
    idx3 = idx.reshape(nw, n_steps, _CHUNK)
    mesh = plsc.VectorSubcoreMesh(core_axis_name="c", subcore_axis_name="s")

    @functools.partial(
        pl.kernel,
        mesh=mesh,
        out_type=jax.ShapeDtypeStruct((b_total, _LPAD), jnp.float32),
        scratch_types=[
            pltpu.VMEM((n_steps, _CHUNK), jnp.int32),
            pltpu.VMEM((_NBUF, _CHUNK, _LPAD), jnp.float32),
            pltpu.SemaphoreType.DMA,
        ],
        compiler_params=pltpu.CompilerParams(use_tc_tiling_on_sc=False),
    )
    def gather_kernel(t_hbm, idx_hbm, out_hbm, idx_v, rows_v, sem):
        wid = lax.axis_index("s") * nc + lax.axis_index("c")
        base = wid * b_per_w
        pltpu.sync_copy(idx_hbm.at[wid], idx_v)

        def group(g, carry):
            copies = []
            for u in range(_NBUF):
                j = g * _NBUF + u
                copies.append(
                    pltpu.async_copy(t_hbm.at[idx_v.at[j]], rows_v.at[u], sem)
                )
            for u in range(_NBUF):
                copies[u].wait()
            for u in range(_NBUF):
                j = g * _NBUF + u
                pltpu.sync_copy(
                    rows_v.at[u], out_hbm.at[pl.ds(base + j * _CHUNK, _CHUNK)]
                )
            return carry

        lax.fori_loop(0, n_groups, group, 0)

    return gather_kernel(table, idx3)


def kernel(input, emb_table, W, b):
    batch, hist = input.shape
    labels = W.shape[1]
    wp = jnp.zeros((_EMBED, _LPAD), jnp.float32).at[:, :labels].set(W)
    bp = jnp.zeros((_LPAD,), jnp.float32).at[:labels].set(b)
    eye = jnp.eye(_PACK, dtype=jnp.float32)
    w8 = jnp.einsum("pq,kn->pkqn", eye, wp).reshape(_PACK * _EMBED, _PACK * _LPAD)
    b8 = jnp.tile(bp, _PACK)[None, :]
    tbl8 = emb_table.reshape(_VOCAB // _PACK, _PACK * _EMBED)

    label_table = _transform_table(tbl8, w8, b8).reshape(_VOCAB, _LPAD)

    idx = input.reshape(-1).astype(jnp.int32)
    gathered = _sc_gather(label_table, idx)
    return gathered[:, :labels].reshape(batch, hist, labels)
